# baseline (speedup 1.0000x reference)
import jax
import jax.numpy as jnp
from jax import lax
from jax.experimental import pallas as pl
from jax.experimental.pallas import tpu as pltpu
from jax.experimental.pallas import tpu_sc as plsc

VOCAB = 100000
DIM = 128
BATCH = 4096
SEQ = 200

_info = plsc.get_sparse_core_info()
_NC, _NS = _info.num_cores, _info.num_subcores
NW = _NC * _NS

B = BATCH * SEQ
B_PER_W = B // NW
CHUNK = 128
NCHUNK = B_PER_W // CHUNK
L = 16

SBLK = 2048
SBITS = 11
NBLK = -(-VOCAB // SBLK)
NBLK_PAD = 50
LAST_W = VOCAB - SBLK
LAST_OFF = (NBLK - 1) * SBLK - LAST_W
NCELL = NBLK_PAD * L
NENT = B_PER_W + NBLK * CHUNK
DUMMY_POS = 0x7FFF
DUMMY_ENT = DUMMY_POS << SBITS
NVEC = B_PER_W // L // 8


def _body(x_hbm, table_hbm, out_hbm,
          idx_v, rows0, rows1, off_stage, pos_stage, hist, cur, bucket, spm,
          gsem0, gsem1, wsem0, wsem1, lsem):
    rows = (rows0, rows1)
    gsems = (gsem0, gsem1)
    wsems = (wsem0, wsem1)
    sid = lax.axis_index("s")
    wid = sid * _NC + lax.axis_index("c")
    base = wid * B_PER_W

    pltpu.sync_copy(x_hbm.at[wid], idx_v)

    def zed(i, c):
        hist[pl.ds(i * L, L)] = jnp.zeros((L,), jnp.int32)
        return c
    lax.fori_loop(0, NCELL // L, zed, 0)

    def dum(i, c):
        bucket[pl.ds(i * L, L)] = jnp.full((L,), DUMMY_ENT, jnp.int32)
        return c
    lax.fori_loop(0, NENT // L, dum, 0)

    def p1(j, c):
        iota = lax.iota(jnp.int32, L)
        for s in range(8):
            idx = idx_v[j, pl.ds(s * L, L)]
            cell = (idx >> SBITS) * L + iota
            cnt = plsc.load_gather(hist, [cell])
            plsc.store_scatter(hist, [cell], cnt + 1)
        return c
    lax.fori_loop(0, NVEC, p1, 0)

    def mkbase(k, pbase):
        row = hist[pl.ds(k * L, L)]
        excl = plsc.cumsum(row) - row
        cur[pl.ds(k * L, L)] = pbase + excl
        nch = (jnp.sum(row) + (CHUNK - 1)) // CHUNK
        return pbase + nch * CHUNK
    lax.fori_loop(0, NBLK, mkbase, jnp.int32(0))

    def p2(j, c):
        iota = lax.iota(jnp.int32, L)
        for s in range(8):
            idx = idx_v[j, pl.ds(s * L, L)]
            bid = idx >> SBITS
            off = (idx & (SBLK - 1)) + jnp.where(
                bid == NBLK - 1, jnp.int32(LAST_OFF), jnp.int32(0))
            pos = j * CHUNK + s * L + iota
            ent = off | (pos << SBITS)
            cell = bid * L + iota
            cnt = plsc.load_gather(cur, [cell])
            plsc.store_scatter(bucket, [cnt], ent)
            plsc.store_scatter(cur, [cell], cnt + 1)
        return c
    lax.fori_loop(0, NVEC, p2, 0)

    for b in range(2):
        pltpu.async_copy(rows[b], out_hbm.at[pl.ds(B, CHUNK)], wsems[b])

    @pl.when(sid == 0)
    def _():
        pltpu.async_copy(table_hbm.at[pl.ds(0, SBLK)],
                         spm.at[pl.ds(0, SBLK)], lsem)
        pltpu.make_async_copy(table_hbm.at[pl.ds(0, SBLK)],
                              spm.at[pl.ds(0, SBLK)], lsem).wait()
    plsc.subcore_barrier()

    def proc_block(pb, nchunks, half):
        sbias = jnp.int32(half * SBLK)

        def chunk_pair(cq, c):
            for u in range(2):
                ch = cq * 2 + u

                @pl.when(ch < nchunks)
                def _():
                    pltpu.make_async_copy(
                        rows[u], out_hbm.at[pl.ds(B, CHUNK)],
                        wsems[u]).wait()
                    iota = lax.iota(jnp.int32, L)
                    ebase = pb + ch * CHUNK
                    for s in range(8):
                        e = bucket[pl.ds(ebase + s * L, L)]
                        off = (e & (SBLK - 1)) + sbias
                        pos = e >> SBITS
                        dm = pos == DUMMY_POS
                        ab = jnp.where(dm, B + iota, base + pos)
                        off_stage[u, pl.ds(s * L, L)] = off
                        pos_stage[u, pl.ds(s * L, L)] = ab

                rp = (u + 1) % 2

                @pl.when((ch >= 1) & (ch <= nchunks))
                def _():
                    pltpu.async_copy(
                        spm.at[off_stage.at[rp]], rows[rp], gsems[rp])
                    pltpu.make_async_copy(
                        spm.at[off_stage.at[rp]], rows[rp], gsems[rp]).wait()
                    pltpu.async_copy(
                        rows[rp], out_hbm.at[pos_stage.at[rp]], wsems[rp])
            return c
        lax.fori_loop(0, (nchunks + 2) // 2, chunk_pair, 0)

    def outer(ko, pbase):
        for kk in range(2):
            k = ko * 2 + kk
            row = hist[pl.ds(k * L, L)]
            nch = (jnp.sum(row) + (CHUNK - 1)) // CHUNK
            inb = k < NBLK

            @pl.when(inb)
            def _():
                @pl.when((sid == 0) & (k + 1 < NBLK))
                def _():
                    wstart = jnp.minimum((k + 1) * SBLK, LAST_W)
                    pltpu.async_copy(
                        table_hbm.at[pl.ds(wstart, SBLK)],
                        spm.at[pl.ds(((kk + 1) % 2) * SBLK, SBLK)], lsem)

                proc_block(pbase, nch, kk)

                @pl.when((sid == 0) & (k + 1 < NBLK))
                def _():
                    wstart = jnp.minimum((k + 1) * SBLK, LAST_W)
                    pltpu.make_async_copy(
                        table_hbm.at[pl.ds(wstart, SBLK)],
                        spm.at[pl.ds(((kk + 1) % 2) * SBLK, SBLK)],
                        lsem).wait()
                plsc.subcore_barrier()

            pbase = pbase + jnp.where(inb, nch * CHUNK, 0)
        return pbase
    lax.fori_loop(0, (NBLK + 1) // 2, outer, jnp.int32(0))

    for b in range(2):
        pltpu.make_async_copy(
            rows[b], out_hbm.at[pl.ds(B, CHUNK)], wsems[b]).wait()


def kernel(x, table):
    mesh = plsc.VectorSubcoreMesh(core_axis_name="c", subcore_axis_name="s")
    x_blocks = x.reshape(NW, NCHUNK, CHUNK).astype(jnp.int32)
    flat = pl.kernel(
        _body,
        out_type=jax.ShapeDtypeStruct((B + CHUNK, DIM), jnp.float32),
        mesh=mesh,
        compiler_params=pltpu.CompilerParams(needs_layout_passes=False),
        scratch_types=(
            [pltpu.VMEM((NCHUNK, CHUNK), jnp.int32)]
            + [pltpu.VMEM((CHUNK, DIM), jnp.float32)] * 2
            + [pltpu.VMEM((2, CHUNK), jnp.int32)] * 2
            + [pltpu.VMEM((NCELL,), jnp.int32)]
            + [pltpu.VMEM((NCELL,), jnp.int32)]
            + [pltpu.VMEM((NENT,), jnp.int32)]
            + [pltpu.VMEM_SHARED((2 * SBLK, DIM), jnp.float32)]
            + [pltpu.SemaphoreType.DMA] * 5
        ),
    )(x_blocks, table)
    return flat[:B].reshape(BATCH, SEQ, DIM)


# speedup vs baseline: 2.4827x; 2.4827x over previous
import jax
import jax.numpy as jnp
from jax import lax
from jax.experimental import pallas as pl
from jax.experimental.pallas import tpu as pltpu
from jax.experimental.pallas import tpu_sc as plsc

VOCAB = 100000
DIM = 128
BATCH = 4096
SEQ = 200

_info = plsc.get_sparse_core_info()
_NC, _NS = _info.num_cores, _info.num_subcores
NW = _NC * _NS

B = BATCH * SEQ
B_PER_W = B // NW
CHUNK = 128
NCHUNK = B_PER_W // CHUNK
NBUF = 2


def _gather_body(x_hbm, table_hbm, out_hbm, idx_v,
                 rows0, rows1, spm,
                 gsem0, gsem1, csem0, csem1, dsem0, dsem1):
    rows = (rows0, rows1)
    gsems = (gsem0, gsem1)
    csems = (csem0, csem1)
    dsems = (dsem0, dsem1)
    sid = lax.axis_index("s")
    wid = sid * _NC + lax.axis_index("c")
    pltpu.sync_copy(x_hbm.at[wid], idx_v)
    base = wid * B_PER_W

    def fire_g(j, b):
        pltpu.async_copy(table_hbm.at[idx_v.at[j]], rows[b], gsems[b])

    def wait_g(j, b):
        pltpu.make_async_copy(
            table_hbm.at[idx_v.at[j]], rows[b], gsems[b]).wait()

    for b in range(NBUF):
        fire_g(b, b)

    def outer(jo, carry):
        for b in range(NBUF):
            j = jo * NBUF + b
            my_spm = spm.at[sid, b]
            wait_g(j, b)

            def wait_d():
                pltpu.make_async_copy(
                    my_spm, out_hbm.at[pl.ds(base, CHUNK)], dsems[b]).wait()
            pl.when(jo > 0)(wait_d)

            pltpu.async_copy(rows[b], my_spm, csems[b])
            pltpu.make_async_copy(rows[b], my_spm, csems[b]).wait()
            pltpu.async_copy(
                my_spm, out_hbm.at[pl.ds(base + j * CHUNK, CHUNK)], dsems[b])

            def next_g():
                fire_g(j + NBUF, b)
            pl.when(jo < (NCHUNK // NBUF) - 1)(next_g)
        return carry

    lax.fori_loop(0, NCHUNK // NBUF, outer, 0)

    for b in range(NBUF):
        pltpu.make_async_copy(
            spm.at[sid, b], out_hbm.at[pl.ds(base, CHUNK)], dsems[b]).wait()


def kernel(x, table):
    mesh = plsc.VectorSubcoreMesh(core_axis_name="c", subcore_axis_name="s")
    x_blocks = x.reshape(NW, NCHUNK, CHUNK).astype(jnp.int32)
    flat = pl.kernel(
        _gather_body,
        out_type=jax.ShapeDtypeStruct((B, DIM), jnp.float32),
        mesh=mesh,
        scratch_types=(
            [pltpu.VMEM((NCHUNK, CHUNK), jnp.int32)]
            + [pltpu.VMEM((CHUNK, DIM), jnp.float32)] * NBUF
            + [pltpu.VMEM_SHARED((_NS, NBUF, CHUNK, DIM), jnp.float32)]
            + [pltpu.SemaphoreType.DMA] * (3 * NBUF)
        ),
    )(x_blocks, table)
    return flat.reshape(BATCH, SEQ, DIM)
